# baseline (speedup 1.0000x reference)
import jax
import jax.numpy as jnp
from jax.experimental import pallas as pl
from jax.experimental.pallas import tpu as pltpu
from jax.experimental.pallas import tpu_sc as plsc

DIM = 192
TIME = 32
HW = 1024
D6 = DIM // 6
DSH = 2 * D6
S_BLK = 4
NLANE = 16
NVEC = DIM // NLANE
NTEC = 32
CHUNKS = 8
CHROWS = HW // CHUNKS
GROUPS = NTEC // CHUNKS
N_ITEMS = 64
ITEMS_PER_GROUP = N_ITEMS // GROUPS


def _tc_cos_body(tidx_ref, ttab_c_ref, spat_c_ref, cos_ref):
    b = pl.program_id(0)
    j = pl.program_id(1)
    spat_c = spat_c_ref[...]
    for u in range(S_BLK):
        idx = tidx_ref[b, j * S_BLK + u]
        cos_ref[0, pl.ds(u * HW, HW), :] = spat_c + ttab_c_ref[pl.ds(idx, 1), :]


def _tc_cos(t_idxs, ttab_c, spat_c, B, S):
    grid_spec = pltpu.PrefetchScalarGridSpec(
        num_scalar_prefetch=1,
        grid=(B, S // S_BLK),
        in_specs=[
            pl.BlockSpec((TIME, DIM), lambda b, s, tidx: (0, 0)),
            pl.BlockSpec((HW, DIM), lambda b, s, tidx: (0, 0)),
        ],
        out_specs=[
            pl.BlockSpec((1, S_BLK * HW, DIM), lambda b, s, tidx: (b, s, 0)),
        ],
    )
    (cos,) = pl.pallas_call(
        _tc_cos_body,
        grid_spec=grid_spec,
        out_shape=[jax.ShapeDtypeStruct((B, S * HW, DIM), jnp.float32)],
        compiler_params=pltpu.CompilerParams(
            dimension_semantics=("parallel", "parallel")),
    )(t_idxs.astype(jnp.int32), ttab_c, spat_c)
    return cos


def _sc_sin_body(ttab_hbm, spat_hbm, tidx_hbm, o_hbm,
                 spat_buf, ttab_buf, tidx_buf, obuf, sem_in, sem_out):
    c = jax.lax.axis_index("c")
    s = jax.lax.axis_index("s")
    wid = s * 2 + c
    chunk = jax.lax.rem(wid, CHUNKS)
    group = jax.lax.div(wid, CHUNKS)

    pltpu.async_copy(spat_hbm.at[pl.ds(chunk * CHROWS, CHROWS), :],
                     spat_buf, sem_in).wait()
    pltpu.async_copy(ttab_hbm, ttab_buf, sem_in).wait()
    pltpu.async_copy(tidx_hbm, tidx_buf, sem_in).wait()

    base_item = group * ITEMS_PER_GROUP
    idx_vec = tidx_buf[pl.ds(base_item, ITEMS_PER_GROUP)]

    def dst(item):
        return o_hbm.at[pl.ds(item * HW + chunk * CHROWS, CHROWS), :]

    for k in range(ITEMS_PER_GROUP):
        slot = k % 2
        item = base_item + k
        if k >= 2:
            pltpu.make_async_copy(obuf.at[slot], dst(item - 2),
                                  sem_out.at[slot]).wait()
        troff = idx_vec[k] * DIM
        trows = [ttab_buf[pl.ds(troff + NLANE * j, NLANE)] for j in range(NVEC)]

        @pl.loop(0, CHROWS)
        def _(r):
            for j in range(NVEC):
                obuf[slot, r, pl.ds(NLANE * j, NLANE)] = (
                    spat_buf[r, pl.ds(NLANE * j, NLANE)] + trows[j])

        pltpu.async_copy(obuf.at[slot], dst(item), sem_out.at[slot])

    for k in range(ITEMS_PER_GROUP - 2, ITEMS_PER_GROUP):
        slot = k % 2
        pltpu.make_async_copy(obuf.at[slot], dst(base_item + k),
                              sem_out.at[slot]).wait()


def _sc_sin(ttab_s_flat, spat_s, tidx_flat):
    mesh = plsc.VectorSubcoreMesh(core_axis_name="c", subcore_axis_name="s")
    kern = pl.kernel(
        _sc_sin_body,
        out_type=jax.ShapeDtypeStruct((N_ITEMS * HW, DIM), jnp.float32),
        mesh=mesh,
        scratch_types=[
            pltpu.VMEM((CHROWS, DIM), jnp.float32),
            pltpu.VMEM((TIME * DIM,), jnp.float32),
            pltpu.VMEM((N_ITEMS,), jnp.int32),
            pltpu.VMEM((2, CHROWS, DIM), jnp.float32),
            pltpu.SemaphoreType.DMA,
            pltpu.SemaphoreType.DMA((2,)),
        ],
    )
    return kern(ttab_s_flat, spat_s, tidx_flat)


def kernel(t_idxs, cos_t, sin_t, cos_s, sin_s):
    B, S = t_idxs.shape
    zt = jnp.zeros((TIME, DSH), jnp.float32)
    ttab_c = jnp.concatenate([cos_t, zt, cos_t, zt], axis=1)
    ttab_s = jnp.concatenate([sin_t, zt, sin_t, zt], axis=1)
    zs = jnp.zeros((HW, D6), jnp.float32)
    spat_c = jnp.concatenate([zs, cos_s, zs, cos_s], axis=1)
    spat_s = jnp.concatenate([zs, sin_s, zs, sin_s], axis=1)

    cos = _tc_cos(t_idxs, ttab_c, spat_c, B, S)
    sin = _sc_sin(ttab_s.reshape(-1), spat_s,
                  t_idxs.reshape(-1).astype(jnp.int32))
    return (cos, sin.reshape(B, S * HW, DIM))
